# baseline (speedup 1.0000x reference)
import jax
import jax.numpy as jnp
from jax.experimental import pallas as pl
from jax.experimental.pallas import tpu as pltpu

_B, _H, _W = 8, 512, 512
_RBLK = 128


def _loss_kernel(score_ref, pred_score_ref, geo_ref, pred_geo_ref, edge_ref,
                 out_ref, acc_ref):
    b = pl.program_id(0)
    i = pl.program_id(1)
    ni = pl.num_programs(1)

    @pl.when(i == 0)
    def _init_batch():
        for k in range(5):
            acc_ref[k] = 0.0

    @pl.when((b == 0) & (i == 0))
    def _init_total():
        acc_ref[5] = 0.0

    score = score_ref[0]
    ps = pred_score_ref[0]
    edge = edge_ref[0]
    d = geo_ref[0] - pred_geo_ref[0]
    ad = jnp.abs(d)
    sl1 = jnp.where(ad < 1.0, 0.5 * d * d, ad - 0.5)
    chsum = jnp.sum(sl1, axis=0)
    mask = (score != 0.0).astype(jnp.float32)
    w = mask / (8.0 * edge)

    acc_ref[0] += jnp.sum(score * ps)
    acc_ref[1] += jnp.sum(score)
    acc_ref[2] += jnp.sum(ps)
    acc_ref[3] += jnp.sum(mask)
    acc_ref[4] += jnp.sum(chsum * w)

    @pl.when(i == ni - 1)
    def _finish_batch():
        a, bs, c, dn, e = (acc_ref[0], acc_ref[1], acc_ref[2],
                           acc_ref[3], acc_ref[4])
        dice = 1.0 - 2.0 * a / (bs + c)
        geo_loss = e / jnp.maximum(dn, 1.0)
        acc_ref[5] += dice + geo_loss

        @pl.when(b == _B - 1)
        def _finalize():
            out_ref[0] = acc_ref[5] / float(_B)


def kernel(score, pred_score, geo, pred_geo, edge):
    grid = (_B, _H // _RBLK)
    out = pl.pallas_call(
        _loss_kernel,
        grid=grid,
        in_specs=[
            pl.BlockSpec((1, _RBLK, _W), lambda b, i: (b, i, 0)),
            pl.BlockSpec((1, _RBLK, _W), lambda b, i: (b, i, 0)),
            pl.BlockSpec((1, 8, _RBLK, _W), lambda b, i: (b, 0, i, 0)),
            pl.BlockSpec((1, 8, _RBLK, _W), lambda b, i: (b, 0, i, 0)),
            pl.BlockSpec((1, _RBLK, _W), lambda b, i: (b, i, 0)),
        ],
        out_specs=pl.BlockSpec(memory_space=pltpu.SMEM),
        out_shape=jax.ShapeDtypeStruct((1,), jnp.float32),
        scratch_shapes=[pltpu.SMEM((6,), jnp.float32)],
    )(score, pred_score, geo, pred_geo, edge)
    return out[0]


# speedup vs baseline: 1.1500x; 1.1500x over previous
import jax
import jax.numpy as jnp
from jax.experimental import pallas as pl
from jax.experimental.pallas import tpu as pltpu

_B, _H, _W = 8, 512, 512
_RBLK = 128


def _loss_kernel(score_ref, pred_score_ref, geo_ref, pred_geo_ref, edge_ref,
                 out_ref, acc_ref):
    b = pl.program_id(0)
    i = pl.program_id(1)
    ni = pl.num_programs(1)

    @pl.when(i == 0)
    def _init_batch():
        for k in range(5):
            acc_ref[k] = 0.0

    @pl.when((b == 0) & (i == 0))
    def _init_total():
        acc_ref[5] = 0.0

    score = score_ref[0]
    ps = pred_score_ref[0]
    edge = edge_ref[0]
    x = geo_ref[0] - pred_geo_ref[0]
    y = jnp.clip(x, -1.0, 1.0)
    sl1 = y * (x - 0.5 * y)
    chsum = jnp.sum(sl1, axis=0)
    mask = (score != 0.0).astype(jnp.float32)
    w = mask * (0.125 / edge)

    acc_ref[0] += jnp.sum(score * ps)
    acc_ref[1] += jnp.sum(score)
    acc_ref[2] += jnp.sum(ps)
    acc_ref[3] += jnp.sum(mask)
    acc_ref[4] += jnp.sum(chsum * w)

    @pl.when(i == ni - 1)
    def _finish_batch():
        a, bs, c, dn, e = (acc_ref[0], acc_ref[1], acc_ref[2],
                           acc_ref[3], acc_ref[4])
        dice = 1.0 - 2.0 * a / (bs + c)
        geo_loss = e / jnp.maximum(dn, 1.0)
        acc_ref[5] += dice + geo_loss

        @pl.when(b == _B - 1)
        def _finalize():
            out_ref[0] = acc_ref[5] / float(_B)


def kernel(score, pred_score, geo, pred_geo, edge):
    grid = (_B, _H // _RBLK)
    out = pl.pallas_call(
        _loss_kernel,
        grid=grid,
        in_specs=[
            pl.BlockSpec((1, _RBLK, _W), lambda b, i: (b, i, 0)),
            pl.BlockSpec((1, _RBLK, _W), lambda b, i: (b, i, 0)),
            pl.BlockSpec((1, 8, _RBLK, _W), lambda b, i: (b, 0, i, 0)),
            pl.BlockSpec((1, 8, _RBLK, _W), lambda b, i: (b, 0, i, 0)),
            pl.BlockSpec((1, _RBLK, _W), lambda b, i: (b, i, 0)),
        ],
        out_specs=pl.BlockSpec(memory_space=pltpu.SMEM),
        out_shape=jax.ShapeDtypeStruct((1,), jnp.float32),
        scratch_shapes=[pltpu.SMEM((6,), jnp.float32)],
    )(score, pred_score, geo, pred_geo, edge)
    return out[0]


# speedup vs baseline: 1.3264x; 1.1534x over previous
import jax
import jax.numpy as jnp
from jax.experimental import pallas as pl
from jax.experimental.pallas import tpu as pltpu

_B, _H, _W = 8, 512, 512
_RBLK = 256


def _loss_kernel(score_ref, pred_score_ref, geo_ref, pred_geo_ref, edge_ref,
                 out_ref, acc_ref):
    b = pl.program_id(0)
    i = pl.program_id(1)
    ni = pl.num_programs(1)

    @pl.when(i == 0)
    def _init_batch():
        for k in range(5):
            acc_ref[k] = 0.0

    @pl.when((b == 0) & (i == 0))
    def _init_total():
        acc_ref[5] = 0.0

    score = score_ref[0]
    ps = pred_score_ref[0]
    edge = edge_ref[0]
    x = geo_ref[0] - pred_geo_ref[0]
    y = jnp.clip(x, -1.0, 1.0)
    sl1 = y * (x - 0.5 * y)
    chsum = jnp.sum(sl1, axis=0)
    mask = (score != 0.0).astype(jnp.float32)
    w = mask * (0.125 / edge)

    acc_ref[0] += jnp.sum(score * ps)
    acc_ref[1] += jnp.sum(score)
    acc_ref[2] += jnp.sum(ps)
    acc_ref[3] += jnp.sum(mask)
    acc_ref[4] += jnp.sum(chsum * w)

    @pl.when(i == ni - 1)
    def _finish_batch():
        a, bs, c, dn, e = (acc_ref[0], acc_ref[1], acc_ref[2],
                           acc_ref[3], acc_ref[4])
        dice = 1.0 - 2.0 * a / (bs + c)
        geo_loss = e / jnp.maximum(dn, 1.0)
        acc_ref[5] += dice + geo_loss

        @pl.when(b == _B - 1)
        def _finalize():
            out_ref[0] = acc_ref[5] / float(_B)


def kernel(score, pred_score, geo, pred_geo, edge):
    grid = (_B, _H // _RBLK)
    out = pl.pallas_call(
        _loss_kernel,
        grid=grid,
        in_specs=[
            pl.BlockSpec((1, _RBLK, _W), lambda b, i: (b, i, 0)),
            pl.BlockSpec((1, _RBLK, _W), lambda b, i: (b, i, 0)),
            pl.BlockSpec((1, 8, _RBLK, _W), lambda b, i: (b, 0, i, 0)),
            pl.BlockSpec((1, 8, _RBLK, _W), lambda b, i: (b, 0, i, 0)),
            pl.BlockSpec((1, _RBLK, _W), lambda b, i: (b, i, 0)),
        ],
        out_specs=pl.BlockSpec(memory_space=pltpu.SMEM),
        out_shape=jax.ShapeDtypeStruct((1,), jnp.float32),
        scratch_shapes=[pltpu.SMEM((6,), jnp.float32)],
    )(score, pred_score, geo, pred_geo, edge)
    return out[0]
